# baseline (speedup 1.0000x reference)
import functools

import jax
import jax.numpy as jnp
from jax import lax
from jax.experimental import pallas as pl
from jax.experimental.pallas import tpu as pltpu
from jax.experimental.pallas import tpu_sc as plsc

N = 10000
NP = 10240
E = 320000
D = 128
NQ = 4
HD = D // NQ
NH = 3
NEG = 0.2
EV = 22754

NC = 2
NS = 16
NW = NC * NS
EPW = E // NW
K = 80
NG = K // 16
NCHUNK = EPW // K
RPT = NP // NS
DW = NP * 4
DWPT = DW // NS
NPAIR = NG * NH
PPS = 5


def _dense_body(x_ref, wlin_ref, wh_ref, asrc_ref, adst_ref,
                hcat_ref, alpha_ref):
    xb = jnp.dot(x_ref[...], wlin_ref[...], preferred_element_type=jnp.float32)
    feats = []
    for h in range(NH):
        feats.append(jnp.dot(xb, wh_ref[h], preferred_element_type=jnp.float32))
    quarters = [
        jnp.concatenate([f[:, q * HD:(q + 1) * HD] for f in feats], axis=1)
        for q in range(NQ)
    ]
    hcat_ref[...] = jnp.stack(quarters, axis=0)
    cols = []
    for h in range(NH):
        cols.append(jnp.sum(feats[h] * asrc_ref[h][None, :], axis=1, keepdims=True))
    for h in range(NH):
        cols.append(jnp.sum(feats[h] * adst_ref[h][None, :], axis=1, keepdims=True))
    alpha_ref[...] = jnp.concatenate(cols, axis=1)


def _dense(x, W_lin, W_heads, a_src, a_dst):
    BN = 1000
    return pl.pallas_call(
        _dense_body,
        grid=(N // BN,),
        in_specs=[
            pl.BlockSpec((BN, D), lambda i: (i, 0)),
            pl.BlockSpec((D, D), lambda i: (0, 0)),
            pl.BlockSpec((NH, D, D), lambda i: (0, 0, 0)),
            pl.BlockSpec((NH, D), lambda i: (0, 0)),
            pl.BlockSpec((NH, D), lambda i: (0, 0)),
        ],
        out_specs=[
            pl.BlockSpec((NQ, BN, NH * HD), lambda i: (0, i, 0)),
            pl.BlockSpec((BN, 6), lambda i: (i, 0)),
        ],
        out_shape=[
            jax.ShapeDtypeStruct((NQ, N, NH * HD), jnp.float32),
            jax.ShapeDtypeStruct((N, 6), jnp.float32),
        ],
    )(x, W_lin, W_heads, a_src, a_dst)


def _etsplit_body(et_ref, cat_ref):
    v = et_ref[...]
    cat_ref[...] = jnp.stack(
        [v[:, q * HD:(q + 1) * HD] for q in range(NQ)], axis=0)


def _etsplit(edge_table):
    BV = 1024
    return pl.pallas_call(
        _etsplit_body,
        grid=(pl.cdiv(EV, BV),),
        in_specs=[pl.BlockSpec((BV, D), lambda i: (i, 0))],
        out_specs=pl.BlockSpec((NQ, BV, HD), lambda i: (0, i, 0)),
        out_shape=jax.ShapeDtypeStruct((NQ, EV, HD), jnp.float32),
    )(edge_table)


def _mesh():
    return plsc.VectorSubcoreMesh(core_axis_name="c", subcore_axis_name="s")


def _pass1_body(src_hbm, dst_hbm, alpha_hbm,
                e0_hbm, e1_hbm, e2_hbm, denp_hbm,
                atbl, srcbs, dstbs, e0b, e1b, e2b,
                evbss, ixbss, zb, dsh, isems, ssems):
    cid = lax.axis_index("c")
    sid = lax.axis_index("s")
    wid = sid * NC + cid
    base = wid * EPW
    ebufs = (e0b, e1b, e2b)
    ehbms = (e0_hbm, e1_hbm, e2_hbm)

    zero16 = jnp.zeros((16,), jnp.float32)

    def zrow(i, _):
        zb[pl.ds(i * 16, 16)] = zero16
        return 0
    lax.fori_loop(0, DWPT // 2 // 16, zrow, 0)

    pltpu.sync_copy(zb, dsh.at[pl.ds(sid * DWPT, DWPT // 2)])
    pltpu.sync_copy(zb, dsh.at[pl.ds(sid * DWPT + DWPT // 2, DWPT // 2)])
    plsc.subcore_barrier()

    pltpu.sync_copy(alpha_hbm, atbl)

    def issue_idx(c, p):
        b = base + c * K
        pltpu.async_copy(src_hbm.at[pl.ds(b, K)], srcbs[p], isems[p])
        pltpu.async_copy(dst_hbm.at[pl.ds(b, K)], dstbs[p], isems[p])

    def wait_idx(p):
        pltpu.make_async_copy(src_hbm.at[pl.ds(0, K)], srcbs[p], isems[p]).wait()
        pltpu.make_async_copy(dst_hbm.at[pl.ds(0, K)], dstbs[p], isems[p]).wait()

    def wait_scat(p):
        for h in range(NH):
            pltpu.make_async_copy(
                evbss[p][h], dsh.at[ixbss[p][h]], ssems[p]).wait()

    def step(c, p, first, last):
        wait_idx(p)
        if not first:
            wait_scat(p)

        def comp(g, _):
            sv6 = srcbs[p][pl.ds(g * 16, 16)] * 6
            dv = dstbs[p][pl.ds(g * 16, 16)]
            dv6 = dv * 6
            dv4 = dv * 4
            for h in range(NH):
                a_s = plsc.load_gather(atbl, [sv6 + h])
                a_d = plsc.load_gather(atbl, [dv6 + (3 + h)])
                z = a_s + a_d
                ev = jnp.exp(jnp.maximum(z, NEG * z))
                ebufs[h][pl.ds(c * K + g * 16, 16)] = ev
                evbss[p][h][pl.ds(g * 16, 16)] = ev
                ixbss[p][h][pl.ds(g * 16, 16)] = dv4 + h
            return 0
        lax.fori_loop(0, NG, comp, 0)
        for h in range(NH):
            pltpu.async_copy(evbss[p][h], dsh.at[ixbss[p][h]], ssems[p],
                             add=True)
        if not last:
            @pl.when(c + 2 < NCHUNK)
            def _():
                issue_idx(c + 2, p)

    issue_idx(0, 0)
    issue_idx(1, 1)
    step(0, 0, True, False)
    step(1, 1, True, False)

    def loop(i, _):
        c = 2 * i
        step(c, 0, False, False)
        step(c + 1, 1, False, False)
        return 0

    lax.fori_loop(1, (NCHUNK - 1) // 2, loop, 0)
    step(NCHUNK - 1, 0, False, True)
    wait_scat(1)
    wait_scat(0)

    for h in range(NH):
        pltpu.sync_copy(ebufs[h], ehbms[h].at[pl.ds(base, EPW)])

    plsc.subcore_barrier()
    pltpu.sync_copy(dsh.at[pl.ds(sid * DWPT, DWPT)],
                    denp_hbm.at[pl.ds(cid * DW + sid * DWPT, DWPT)])


def _pass1(src, dst, alpha_flat):
    f = functools.partial(
        pl.kernel,
        out_type=(
            jax.ShapeDtypeStruct((E,), jnp.float32),
            jax.ShapeDtypeStruct((E,), jnp.float32),
            jax.ShapeDtypeStruct((E,), jnp.float32),
            jax.ShapeDtypeStruct((NC * DW,), jnp.float32),
        ),
        mesh=_mesh(),
        scratch_types=[
            pltpu.VMEM((N * 6,), jnp.float32),
            [pltpu.VMEM((K,), jnp.int32) for _ in range(2)],
            [pltpu.VMEM((K,), jnp.int32) for _ in range(2)],
            pltpu.VMEM((EPW,), jnp.float32),
            pltpu.VMEM((EPW,), jnp.float32),
            pltpu.VMEM((EPW,), jnp.float32),
            [[pltpu.VMEM((PPS * 16,), jnp.float32) for _ in range(3)]
             for _ in range(2)],
            [[pltpu.VMEM((PPS * 16,), jnp.int32) for _ in range(3)]
             for _ in range(2)],
            pltpu.VMEM((DWPT // 2,), jnp.float32),
            pltpu.VMEM_SHARED((DW,), jnp.float32),
            [pltpu.SemaphoreType.DMA for _ in range(2)],
            [pltpu.SemaphoreType.DMA for _ in range(2)],
        ],
        compiler_params=pltpu.CompilerParams(needs_layout_passes=False),
    )(_pass1_body)
    return f(src, dst, alpha_flat)


def _rden_body(denp_ref, rden_ref):
    s = denp_ref[0:DW] + denp_ref[DW:2 * DW]
    rden_ref[...] = 1.0 / (s + 1e-16)


def _rden(denp):
    return pl.pallas_call(
        _rden_body,
        out_shape=jax.ShapeDtypeStruct((DW,), jnp.float32),
    )(denp)


def _pass2_body(src_hbm, dst_hbm, ewi_hbm, e0_hbm, e1_hbm, e2_hbm,
                hcat_hbm, etcat_hbm, rden_hbm, outp_hbm,
                rtbl, srcbs, dstbs, ewibs, ebss, sdst, w0b, w1b, w2b,
                hbufs, ewbufs, msgbuf, osh, isems, gsems, ssem):
    cid = lax.axis_index("c")
    sid = lax.axis_index("s")
    wid = sid * NC + cid
    base = wid * EPW
    wbufs = (w0b, w1b, w2b)
    ehbms = (e0_hbm, e1_hbm, e2_hbm)

    zero16 = jnp.zeros((16,), jnp.float32)

    def zmsg():
        def zrow(i, _):
            for q in range(HD // 16):
                msgbuf[i, pl.ds(q * 16, 16)] = zero16
            return 0
        lax.fori_loop(0, K, zrow, 0)

    def zosh():
        for j in range(RPT // K):
            pltpu.sync_copy(msgbuf, osh.at[pl.ds(sid * RPT + j * K, K)])

    zmsg()
    zosh()
    plsc.subcore_barrier()

    pltpu.sync_copy(rden_hbm, rtbl)

    def issue_idx(c, p):
        b = base + c * K
        pltpu.async_copy(src_hbm.at[pl.ds(b, K)], srcbs[p], isems[p])
        pltpu.async_copy(dst_hbm.at[pl.ds(b, K)], dstbs[p], isems[p])
        pltpu.async_copy(ewi_hbm.at[pl.ds(b, K)], ewibs[p], isems[p])
        for h in range(NH):
            pltpu.async_copy(ehbms[h].at[pl.ds(b, K)], ebss[p][h], isems[p])

    def wait_idx(p, half):
        pltpu.make_async_copy(src_hbm.at[pl.ds(0, K)], srcbs[p], isems[p]).wait()
        pltpu.make_async_copy(dst_hbm.at[pl.ds(0, K)], dstbs[p], isems[p]).wait()
        pltpu.make_async_copy(ewi_hbm.at[pl.ds(0, K)], ewibs[p], isems[p]).wait()
        for h in range(NH):
            pltpu.make_async_copy(
                ehbms[h].at[pl.ds(0, K)], ebss[p][h], isems[p]).wait()
        srow = half * N
        erow = half * EV

        def fix(g, _):
            srcbs[p][pl.ds(g * 16, 16)] = srcbs[p][pl.ds(g * 16, 16)] + srow
            ewibs[p][pl.ds(g * 16, 16)] = ewibs[p][pl.ds(g * 16, 16)] + erow
            return 0
        lax.fori_loop(0, NG, fix, 0)

    def issue_gath(p):
        pltpu.async_copy(hcat_hbm.at[srcbs[p]], hbufs[p], gsems[p])
        pltpu.async_copy(etcat_hbm.at[ewibs[p]], ewbufs[p], gsems[p])

    def wait_gath(p):
        pltpu.make_async_copy(hcat_hbm.at[srcbs[p]], hbufs[p], gsems[p]).wait()
        pltpu.make_async_copy(etcat_hbm.at[ewibs[p]], ewbufs[p],
                              gsems[p]).wait()

    def wait_scat():
        pltpu.make_async_copy(msgbuf, osh.at[sdst], ssem).wait()

    def step(c, p, first, last, half):
        wait_gath(p)

        def wcomp(g, _):
            dv = dstbs[p][pl.ds(g * 16, 16)]
            dv4 = dv * 4
            for h in range(NH):
                rd = plsc.load_gather(rtbl, [dv4 + h])
                wbufs[h][pl.ds(g * 16, 16)] = \
                    ebss[p][h][pl.ds(g * 16, 16)] * rd * jnp.float32(1.0 / NH)
            return 0
        lax.fori_loop(0, NG, wcomp, 0)
        if not first:
            wait_scat()

        def cpd(g, _):
            sdst[pl.ds(g * 16, 16)] = dstbs[p][pl.ds(g * 16, 16)]
            return 0
        lax.fori_loop(0, NG, cpd, 0)
        if not last:
            @pl.when(c + 3 < NCHUNK)
            def _():
                issue_idx(c + 3, p)
            pn = (p + 2) % 3
            wait_idx(pn, half)
            issue_gath(pn)
        hbuf = hbufs[p]
        ewbuf = ewbufs[p]

        def mcomp(g, _):
            wv0 = w0b[pl.ds(g * 16, 16)]
            wv1 = w1b[pl.ds(g * 16, 16)]
            wv2 = w2b[pl.ds(g * 16, 16)]
            for kk in range(16):
                k = g * 16 + kk
                w0 = wv0[kk]
                w1 = wv1[kk]
                w2 = wv2[kk]
                for q in range(HD // 16):
                    sl = pl.ds(q * 16, 16)
                    m = (hbuf[k, pl.ds(q * 16, 16)] * w0
                         + hbuf[k, pl.ds(HD + q * 16, 16)] * w1
                         + hbuf[k, pl.ds(2 * HD + q * 16, 16)] * w2)
                    msgbuf[k, sl] = m * ewbuf[k, sl]
            return 0
        lax.fori_loop(0, NG, mcomp, 0)
        pltpu.async_copy(msgbuf, osh.at[sdst], ssem, add=True)

    def run_half(half, _):
        issue_idx(0, 0)
        issue_idx(1, 1)
        issue_idx(2, 2)
        wait_idx(0, half)
        issue_gath(0)
        wait_idx(1, half)
        issue_gath(1)
        step(0, 0, True, False, half)
        step(1, 1, False, False, half)
        step(2, 2, False, False, half)

        def loop(i, _):
            c = 3 * i
            step(c, 0, False, False, half)
            step(c + 1, 1, False, False, half)
            step(c + 2, 2, False, False, half)
            return 0

        lax.fori_loop(1, (NCHUNK - 2) // 3, loop, 0)
        step(NCHUNK - 2, 0, False, True, half)
        step(NCHUNK - 1, 1, False, True, half)
        wait_scat()

        plsc.subcore_barrier()
        pltpu.sync_copy(
            osh.at[pl.ds(sid * RPT, RPT)],
            outp_hbm.at[pl.ds(half * (NC * NP) + cid * NP + sid * RPT, RPT)])
        zmsg()
        zosh()
        plsc.subcore_barrier()
        return 0

    lax.fori_loop(0, NQ, run_half, 0)


def _pass2(src, dst, ewi, e0, e1, e2, Hcat, etcat, rden):
    f = functools.partial(
        pl.kernel,
        out_type=jax.ShapeDtypeStruct((NQ * NC * NP, HD), jnp.float32),
        mesh=_mesh(),
        scratch_types=[
            pltpu.VMEM((DW,), jnp.float32),
            [pltpu.VMEM((K,), jnp.int32) for _ in range(3)],
            [pltpu.VMEM((K,), jnp.int32) for _ in range(3)],
            [pltpu.VMEM((K,), jnp.int32) for _ in range(3)],
            [[pltpu.VMEM((K,), jnp.float32) for _ in range(NH)]
             for _ in range(3)],
            pltpu.VMEM((K,), jnp.int32),
            pltpu.VMEM((K,), jnp.float32),
            pltpu.VMEM((K,), jnp.float32),
            pltpu.VMEM((K,), jnp.float32),
            [pltpu.VMEM((K, NH * HD), jnp.float32) for _ in range(3)],
            [pltpu.VMEM((K, HD), jnp.float32) for _ in range(3)],
            pltpu.VMEM((K, HD), jnp.float32),
            pltpu.VMEM_SHARED((NP, HD), jnp.float32),
            [pltpu.SemaphoreType.DMA for _ in range(3)],
            [pltpu.SemaphoreType.DMA for _ in range(3)],
            pltpu.SemaphoreType.DMA,
        ],
        compiler_params=pltpu.CompilerParams(
            needs_layout_passes=False, use_tc_tiling_on_sc=False),
    )(_pass2_body)
    return f(src, dst, ewi, e0, e1, e2, Hcat, etcat, rden)


def _final_body(outp_ref, out_ref):
    parts = []
    for q in range(NQ):
        b = q * NC * NP
        parts.append(outp_ref[b:b + N, :] + outp_ref[b + NP:b + NP + N, :])
    out_ref[...] = jnp.concatenate(parts, axis=1)


def _final(outp):
    return pl.pallas_call(
        _final_body,
        out_shape=jax.ShapeDtypeStruct((N, D), jnp.float32),
    )(outp)


def kernel(x, edge_index, edge_weight, W_lin, edge_table, W_heads, a_src, a_dst):
    ei = edge_index.astype(jnp.int32)
    src = ei[0]
    dst = ei[1]
    ewi = edge_weight.astype(jnp.int32)
    Hcat, alpha = _dense(x, W_lin, W_heads, a_src, a_dst)
    etcat = _etsplit(edge_table)
    e0, e1, e2, denp = _pass1(src, dst, alpha.reshape(N * 6))
    rden = _rden(denp)
    outp = _pass2(src, dst, ewi, e0, e1, e2,
                  Hcat.reshape(NQ * N, NH * HD),
                  etcat.reshape(NQ * EV, HD), rden)
    return _final(outp)


# speedup vs baseline: 1.0129x; 1.0129x over previous
import functools

import jax
import jax.numpy as jnp
from jax import lax
from jax.experimental import pallas as pl
from jax.experimental.pallas import tpu as pltpu
from jax.experimental.pallas import tpu_sc as plsc

N = 10000
NP = 10240
E = 320000
D = 128
NQ = 4
HD = D // NQ
NH = 3
NEG = 0.2
EV = 22754

NC = 2
NS = 16
NW = NC * NS
EPW = E // NW
K = 80
NG = K // 16
NCHUNK = EPW // K
RPT = NP // NS
DW = NP * 4
DWPT = DW // NS
NPAIR = NG * NH
HW = HD // 2
BMSK = -65536
PPS = 5


def _dense_body(x_ref, wlin_ref, wh_ref, asrc_ref, adst_ref,
                hcat_ref, alpha_ref):
    xb = jnp.dot(x_ref[...], wlin_ref[...], preferred_element_type=jnp.float32)
    feats = []
    for h in range(NH):
        feats.append(jnp.dot(xb, wh_ref[h], preferred_element_type=jnp.float32))
    hcat_ref[...] = jnp.stack(feats, axis=0).astype(jnp.bfloat16)
    cols = []
    for h in range(NH):
        cols.append(jnp.sum(feats[h] * asrc_ref[h][None, :], axis=1, keepdims=True))
    for h in range(NH):
        cols.append(jnp.sum(feats[h] * adst_ref[h][None, :], axis=1, keepdims=True))
    alpha_ref[...] = jnp.concatenate(cols, axis=1)


def _dense(x, W_lin, W_heads, a_src, a_dst):
    BN = 2000
    return pl.pallas_call(
        _dense_body,
        grid=(N // BN,),
        in_specs=[
            pl.BlockSpec((BN, D), lambda i: (i, 0)),
            pl.BlockSpec((D, D), lambda i: (0, 0)),
            pl.BlockSpec((NH, D, D), lambda i: (0, 0, 0)),
            pl.BlockSpec((NH, D), lambda i: (0, 0)),
            pl.BlockSpec((NH, D), lambda i: (0, 0)),
        ],
        out_specs=[
            pl.BlockSpec((NH, BN, D), lambda i: (0, i, 0)),
            pl.BlockSpec((BN, 6), lambda i: (i, 0)),
        ],
        out_shape=[
            jax.ShapeDtypeStruct((NH, N, D), jnp.bfloat16),
            jax.ShapeDtypeStruct((N, 6), jnp.float32),
        ],
    )(x, W_lin, W_heads, a_src, a_dst)


def _etsplit_body(et_ref, cat_ref):
    cat_ref[...] = et_ref[...].astype(jnp.bfloat16)


def _etsplit(edge_table):
    BV = 1024
    return pl.pallas_call(
        _etsplit_body,
        grid=(pl.cdiv(EV, BV),),
        in_specs=[pl.BlockSpec((BV, D), lambda i: (i, 0))],
        out_specs=pl.BlockSpec((BV, D), lambda i: (i, 0)),
        out_shape=jax.ShapeDtypeStruct((EV, D), jnp.bfloat16),
    )(edge_table)


def _mesh():
    return plsc.VectorSubcoreMesh(core_axis_name="c", subcore_axis_name="s")


def _pass1_body(src_hbm, dst_hbm, alpha_hbm,
                e0_hbm, e1_hbm, e2_hbm, denp_hbm,
                atbl, srcbs, dstbs, e0b, e1b, e2b,
                evbss, ixbss, zb, dsh, isems, ssems):
    cid = lax.axis_index("c")
    sid = lax.axis_index("s")
    wid = sid * NC + cid
    base = wid * EPW
    ebufs = (e0b, e1b, e2b)
    ehbms = (e0_hbm, e1_hbm, e2_hbm)

    zero16 = jnp.zeros((16,), jnp.float32)

    def zrow(i, _):
        zb[pl.ds(i * 16, 16)] = zero16
        return 0
    lax.fori_loop(0, DWPT // 2 // 16, zrow, 0)

    pltpu.sync_copy(zb, dsh.at[pl.ds(sid * DWPT, DWPT // 2)])
    pltpu.sync_copy(zb, dsh.at[pl.ds(sid * DWPT + DWPT // 2, DWPT // 2)])
    plsc.subcore_barrier()

    pltpu.sync_copy(alpha_hbm, atbl)

    def issue_idx(c, p):
        b = base + c * K
        pltpu.async_copy(src_hbm.at[pl.ds(b, K)], srcbs[p], isems[p])
        pltpu.async_copy(dst_hbm.at[pl.ds(b, K)], dstbs[p], isems[p])

    def wait_idx(p):
        pltpu.make_async_copy(src_hbm.at[pl.ds(0, K)], srcbs[p], isems[p]).wait()
        pltpu.make_async_copy(dst_hbm.at[pl.ds(0, K)], dstbs[p], isems[p]).wait()

    def wait_scat(p):
        for h in range(NH):
            pltpu.make_async_copy(
                evbss[p][h], dsh.at[ixbss[p][h]], ssems[p]).wait()

    def step(c, p, first, last):
        wait_idx(p)
        if not first:
            wait_scat(p)

        def comp(g, _):
            sv6 = srcbs[p][pl.ds(g * 16, 16)] * 6
            dv = dstbs[p][pl.ds(g * 16, 16)]
            dv6 = dv * 6
            dv4 = dv * 4
            for h in range(NH):
                a_s = plsc.load_gather(atbl, [sv6 + h])
                a_d = plsc.load_gather(atbl, [dv6 + (3 + h)])
                z = a_s + a_d
                ev = jnp.exp(jnp.maximum(z, NEG * z))
                ebufs[h][pl.ds(c * K + g * 16, 16)] = ev
                evbss[p][h][pl.ds(g * 16, 16)] = ev
                ixbss[p][h][pl.ds(g * 16, 16)] = dv4 + h
            return 0
        lax.fori_loop(0, NG, comp, 0)
        for h in range(NH):
            pltpu.async_copy(evbss[p][h], dsh.at[ixbss[p][h]], ssems[p],
                             add=True)
        if not last:
            @pl.when(c + 2 < NCHUNK)
            def _():
                issue_idx(c + 2, p)

    issue_idx(0, 0)
    issue_idx(1, 1)
    step(0, 0, True, False)
    step(1, 1, True, False)

    def loop(i, _):
        c = 2 * i
        step(c, 0, False, False)
        step(c + 1, 1, False, False)
        return 0

    lax.fori_loop(1, (NCHUNK - 1) // 2, loop, 0)
    step(NCHUNK - 1, 0, False, True)
    wait_scat(1)
    wait_scat(0)

    for h in range(NH):
        pltpu.sync_copy(ebufs[h], ehbms[h].at[pl.ds(base, EPW)])

    plsc.subcore_barrier()
    pltpu.sync_copy(dsh.at[pl.ds(sid * DWPT, DWPT)],
                    denp_hbm.at[pl.ds(cid * DW + sid * DWPT, DWPT)])


def _pass1(src, dst, alpha_flat):
    f = functools.partial(
        pl.kernel,
        out_type=(
            jax.ShapeDtypeStruct((E,), jnp.float32),
            jax.ShapeDtypeStruct((E,), jnp.float32),
            jax.ShapeDtypeStruct((E,), jnp.float32),
            jax.ShapeDtypeStruct((NC * DW,), jnp.float32),
        ),
        mesh=_mesh(),
        scratch_types=[
            pltpu.VMEM((N * 6,), jnp.float32),
            [pltpu.VMEM((K,), jnp.int32) for _ in range(2)],
            [pltpu.VMEM((K,), jnp.int32) for _ in range(2)],
            pltpu.VMEM((EPW,), jnp.float32),
            pltpu.VMEM((EPW,), jnp.float32),
            pltpu.VMEM((EPW,), jnp.float32),
            [[pltpu.VMEM((PPS * 16,), jnp.float32) for _ in range(3)]
             for _ in range(2)],
            [[pltpu.VMEM((PPS * 16,), jnp.int32) for _ in range(3)]
             for _ in range(2)],
            pltpu.VMEM((DWPT // 2,), jnp.float32),
            pltpu.VMEM_SHARED((DW,), jnp.float32),
            [pltpu.SemaphoreType.DMA for _ in range(2)],
            [pltpu.SemaphoreType.DMA for _ in range(2)],
        ],
        compiler_params=pltpu.CompilerParams(needs_layout_passes=False),
    )(_pass1_body)
    return f(src, dst, alpha_flat)


def _rden_body(denp_ref, rden_ref):
    s = denp_ref[0:DW] + denp_ref[DW:2 * DW]
    rden_ref[...] = 1.0 / (s + 1e-16)


def _rden(denp):
    return pl.pallas_call(
        _rden_body,
        out_shape=jax.ShapeDtypeStruct((DW,), jnp.float32),
    )(denp)


def _pass2_body(src_hbm, dst_hbm, ewi_hbm, e0_hbm, e1_hbm, e2_hbm,
                hcat_hbm, etcat_hbm, rden_hbm, outp_hbm,
                rtbl, srcbs, dstbs, ewibs, ebss, sdst, w0b, w1b, w2b,
                hbufs, ewbufs, msgbuf, osh, isems, gsems, ssem):
    cid = lax.axis_index("c")
    sid = lax.axis_index("s")
    wid = sid * NC + cid
    base = wid * EPW
    wbufs = (w0b, w1b, w2b)
    ehbms = (e0_hbm, e1_hbm, e2_hbm)

    zero16 = jnp.zeros((16,), jnp.float32)

    def zmsg():
        def zrow(i, _):
            for q in range(HD // 16):
                msgbuf[i, pl.ds(q * 16, 16)] = zero16
            return 0
        lax.fori_loop(0, K, zrow, 0)

    def zosh():
        for j in range(RPT // K):
            pltpu.sync_copy(msgbuf, osh.at[pl.ds(sid * RPT + j * K, K)])

    zmsg()
    zosh()
    plsc.subcore_barrier()

    pltpu.sync_copy(rden_hbm, rtbl)

    def issue_idx(c, p):
        b = base + c * K
        pltpu.async_copy(src_hbm.at[pl.ds(b, K)], srcbs[p], isems[p])
        pltpu.async_copy(dst_hbm.at[pl.ds(b, K)], dstbs[p], isems[p])
        pltpu.async_copy(ewi_hbm.at[pl.ds(b, K)], ewibs[p], isems[p])
        for h in range(NH):
            pltpu.async_copy(ehbms[h].at[pl.ds(b, K)], ebss[p][h], isems[p])

    def wait_idx(p, half):
        pltpu.make_async_copy(src_hbm.at[pl.ds(0, K)], srcbs[p], isems[p]).wait()
        pltpu.make_async_copy(dst_hbm.at[pl.ds(0, K)], dstbs[p], isems[p]).wait()
        pltpu.make_async_copy(ewi_hbm.at[pl.ds(0, K)], ewibs[p], isems[p]).wait()
        for h in range(NH):
            pltpu.make_async_copy(
                ehbms[h].at[pl.ds(0, K)], ebss[p][h], isems[p]).wait()
        srow = half * N
        erow = half * EV

        def fix(g, _):
            srcbs[p][pl.ds(g * 16, 16)] = srcbs[p][pl.ds(g * 16, 16)] + srow
            ewibs[p][pl.ds(g * 16, 16)] = ewibs[p][pl.ds(g * 16, 16)] + erow
            return 0
        lax.fori_loop(0, NG, fix, 0)

    def issue_gath(p):
        pltpu.async_copy(hcat_hbm.at[srcbs[p]], hbufs[p], gsems[p])
        pltpu.async_copy(etcat_hbm.at[ewibs[p]], ewbufs[p], gsems[p])

    def wait_gath(p):
        pltpu.make_async_copy(hcat_hbm.at[srcbs[p]], hbufs[p], gsems[p]).wait()
        pltpu.make_async_copy(etcat_hbm.at[ewibs[p]], ewbufs[p],
                              gsems[p]).wait()

    def wait_scat():
        pltpu.make_async_copy(msgbuf, osh.at[sdst], ssem).wait()

    def step(c, p, first, last, half):
        wait_gath(p)

        def wcomp(g, _):
            dv = dstbs[p][pl.ds(g * 16, 16)]
            dv4 = dv * 4
            for h in range(NH):
                rd = plsc.load_gather(rtbl, [dv4 + h])
                wbufs[h][pl.ds(g * 16, 16)] = \
                    ebss[p][h][pl.ds(g * 16, 16)] * rd * jnp.float32(1.0 / NH)
            return 0
        lax.fori_loop(0, NG, wcomp, 0)
        if not first:
            wait_scat()

        def cpd(g, _):
            sdst[pl.ds(g * 16, 16)] = dstbs[p][pl.ds(g * 16, 16)]
            return 0
        lax.fori_loop(0, NG, cpd, 0)
        if not last:
            @pl.when(c + 3 < NCHUNK)
            def _():
                issue_idx(c + 3, p)
            pn = (p + 2) % 3
            wait_idx(pn, half)
            issue_gath(pn)
        hbuf = hbufs[p]
        ewbuf = ewbufs[p]

        def mcomp(g, _):
            wv0 = w0b[pl.ds(g * 16, 16)]
            wv1 = w1b[pl.ds(g * 16, 16)]
            wv2 = w2b[pl.ds(g * 16, 16)]
            for kk in range(16):
                k = g * 16 + kk
                w0 = wv0[kk]
                w1 = wv1[kk]
                w2 = wv2[kk]
                hv = [hbuf[k, pl.ds(h * HW, HW)] for h in range(NH)]
                ev = ewbuf[k, pl.ds(0, HW)]
                me = (plsc.bitcast(hv[0] << 16, jnp.float32) * w0
                      + plsc.bitcast(hv[1] << 16, jnp.float32) * w1
                      + plsc.bitcast(hv[2] << 16, jnp.float32) * w2) \
                    * plsc.bitcast(ev << 16, jnp.float32)
                mo = (plsc.bitcast(hv[0] & BMSK, jnp.float32) * w0
                      + plsc.bitcast(hv[1] & BMSK, jnp.float32) * w1
                      + plsc.bitcast(hv[2] & BMSK, jnp.float32) * w2) \
                    * plsc.bitcast(ev & BMSK, jnp.float32)
                msgbuf[k, pl.ds(0, 16)] = me
                msgbuf[k, pl.ds(16, 16)] = mo
            return 0
        lax.fori_loop(0, NG, mcomp, 0)
        pltpu.async_copy(msgbuf, osh.at[sdst], ssem, add=True)

    def run_half(half, _):
        issue_idx(0, 0)
        issue_idx(1, 1)
        issue_idx(2, 2)
        wait_idx(0, half)
        issue_gath(0)
        wait_idx(1, half)
        issue_gath(1)
        step(0, 0, True, False, half)
        step(1, 1, False, False, half)
        step(2, 2, False, False, half)

        def loop(i, _):
            c = 3 * i
            step(c, 0, False, False, half)
            step(c + 1, 1, False, False, half)
            step(c + 2, 2, False, False, half)
            return 0

        lax.fori_loop(1, (NCHUNK - 2) // 3, loop, 0)
        step(NCHUNK - 2, 0, False, True, half)
        step(NCHUNK - 1, 1, False, True, half)
        wait_scat()

        plsc.subcore_barrier()
        pltpu.sync_copy(
            osh.at[pl.ds(sid * RPT, RPT)],
            outp_hbm.at[pl.ds(half * (NC * NP) + cid * NP + sid * RPT, RPT)])
        zmsg()
        zosh()
        plsc.subcore_barrier()
        return 0

    lax.fori_loop(0, NQ, run_half, 0)


def _pass2(src, dst, ewi, e0, e1, e2, Hcat, etcat, rden):
    f = functools.partial(
        pl.kernel,
        out_type=jax.ShapeDtypeStruct((NQ * NC * NP, HD), jnp.float32),
        mesh=_mesh(),
        scratch_types=[
            pltpu.VMEM((DW,), jnp.float32),
            [pltpu.VMEM((K,), jnp.int32) for _ in range(3)],
            [pltpu.VMEM((K,), jnp.int32) for _ in range(3)],
            [pltpu.VMEM((K,), jnp.int32) for _ in range(3)],
            [[pltpu.VMEM((K,), jnp.float32) for _ in range(NH)]
             for _ in range(3)],
            pltpu.VMEM((K,), jnp.int32),
            pltpu.VMEM((K,), jnp.float32),
            pltpu.VMEM((K,), jnp.float32),
            pltpu.VMEM((K,), jnp.float32),
            [pltpu.VMEM((K, NH * HW), jnp.int32) for _ in range(3)],
            [pltpu.VMEM((K, HW), jnp.int32) for _ in range(3)],
            pltpu.VMEM((K, HD), jnp.float32),
            pltpu.VMEM_SHARED((NP, HD), jnp.float32),
            [pltpu.SemaphoreType.DMA for _ in range(3)],
            [pltpu.SemaphoreType.DMA for _ in range(3)],
            pltpu.SemaphoreType.DMA,
        ],
        compiler_params=pltpu.CompilerParams(
            needs_layout_passes=False, use_tc_tiling_on_sc=False),
    )(_pass2_body)
    return f(src, dst, ewi, e0, e1, e2, Hcat, etcat, rden)


def _final_body(outp_ref, out_ref):
    parts = []
    for q in range(NQ):
        b = q * NC * NP
        parts.append(outp_ref[b:b + N, :] + outp_ref[b + NP:b + NP + N, :])
    out_ref[...] = jnp.concatenate(parts, axis=1)


def _final(outp):
    return pl.pallas_call(
        _final_body,
        out_shape=jax.ShapeDtypeStruct((N, D), jnp.float32),
    )(outp)


def kernel(x, edge_index, edge_weight, W_lin, edge_table, W_heads, a_src, a_dst):
    ei = edge_index.astype(jnp.int32)
    src = ei[0]
    dst = ei[1]
    ewi = edge_weight.astype(jnp.int32)
    Hcat, alpha = _dense(x, W_lin, W_heads, a_src, a_dst)
    etcat = _etsplit(edge_table)
    e0, e1, e2, denp = _pass1(src, dst, alpha.reshape(N * 6))
    rden = _rden(denp)
    hq = Hcat.reshape(NH, N, NQ, HD).transpose(2, 1, 0, 3)
    hi32 = lax.bitcast_convert_type(
        hq.reshape(NQ * N, NH * HW, 2), jnp.int32)
    eq = etcat.reshape(EV, NQ, HD).transpose(1, 0, 2)
    ei32 = lax.bitcast_convert_type(
        eq.reshape(NQ * EV, HW, 2), jnp.int32)
    outp = _pass2(src, dst, ewi, e0, e1, e2, hi32, ei32, rden)
    out_perm = _final(outp)
    return out_perm.reshape(N, NQ, 2, 16).transpose(0, 1, 3, 2).reshape(N, D)


# speedup vs baseline: 1.0166x; 1.0036x over previous
import functools

import jax
import jax.numpy as jnp
from jax import lax
from jax.experimental import pallas as pl
from jax.experimental.pallas import tpu as pltpu
from jax.experimental.pallas import tpu_sc as plsc

N = 10000
NP = 10240
E = 320000
D = 128
NQ = 4
HD = D // NQ
NH = 3
NEG = 0.2
EV = 22754

NC = 2
NS = 16
NW = NC * NS
EPW = E // NW
K = 80
NG = K // 16
NCHUNK = EPW // K
RPT = NP // NS
DW = NP * 4
DWPT = DW // NS
NPAIR = NG * NH
HW = HD // 2
BMSK = -65536
PPS = 5


def _dense_body(x_ref, wlin_ref, wh_ref, asrc_ref, adst_ref,
                hcat_ref, alpha_ref):
    xb = jnp.dot(x_ref[...], wlin_ref[...], preferred_element_type=jnp.float32)
    feats = []
    for h in range(NH):
        feats.append(jnp.dot(xb, wh_ref[h], preferred_element_type=jnp.float32))
    hcat_ref[...] = jnp.stack(feats, axis=0).astype(jnp.bfloat16)
    cols = []
    for h in range(NH):
        cols.append(jnp.sum(feats[h] * asrc_ref[h][None, :], axis=1, keepdims=True))
    for h in range(NH):
        cols.append(jnp.sum(feats[h] * adst_ref[h][None, :], axis=1, keepdims=True))
    alpha_ref[...] = jnp.concatenate(cols, axis=1)


def _dense(x, W_lin, W_heads, a_src, a_dst):
    BN = 2000
    return pl.pallas_call(
        _dense_body,
        grid=(N // BN,),
        in_specs=[
            pl.BlockSpec((BN, D), lambda i: (i, 0)),
            pl.BlockSpec((D, D), lambda i: (0, 0)),
            pl.BlockSpec((NH, D, D), lambda i: (0, 0, 0)),
            pl.BlockSpec((NH, D), lambda i: (0, 0)),
            pl.BlockSpec((NH, D), lambda i: (0, 0)),
        ],
        out_specs=[
            pl.BlockSpec((NH, BN, D), lambda i: (0, i, 0)),
            pl.BlockSpec((BN, 6), lambda i: (i, 0)),
        ],
        out_shape=[
            jax.ShapeDtypeStruct((NH, N, D), jnp.bfloat16),
            jax.ShapeDtypeStruct((N, 6), jnp.float32),
        ],
    )(x, W_lin, W_heads, a_src, a_dst)


def _etsplit_body(et_ref, cat_ref):
    cat_ref[...] = et_ref[...].astype(jnp.bfloat16)


def _etsplit(edge_table):
    BV = 1024
    return pl.pallas_call(
        _etsplit_body,
        grid=(pl.cdiv(EV, BV),),
        in_specs=[pl.BlockSpec((BV, D), lambda i: (i, 0))],
        out_specs=pl.BlockSpec((BV, D), lambda i: (i, 0)),
        out_shape=jax.ShapeDtypeStruct((EV, D), jnp.bfloat16),
    )(edge_table)


def _mesh():
    return plsc.VectorSubcoreMesh(core_axis_name="c", subcore_axis_name="s")


def _pass1_body(src_hbm, dst_hbm, alpha_hbm,
                e0_hbm, e1_hbm, e2_hbm, denp_hbm,
                atbl, srcbs, dstbs, e0b, e1b, e2b,
                evbss, ixbss, zb, dsh, isems, ssems):
    cid = lax.axis_index("c")
    sid = lax.axis_index("s")
    wid = sid * NC + cid
    base = wid * EPW
    ebufs = (e0b, e1b, e2b)
    ehbms = (e0_hbm, e1_hbm, e2_hbm)

    zero16 = jnp.zeros((16,), jnp.float32)

    def zrow(i, _):
        zb[pl.ds(i * 16, 16)] = zero16
        return 0
    lax.fori_loop(0, DWPT // 2 // 16, zrow, 0)

    pltpu.sync_copy(zb, dsh.at[pl.ds(sid * DWPT, DWPT // 2)])
    pltpu.sync_copy(zb, dsh.at[pl.ds(sid * DWPT + DWPT // 2, DWPT // 2)])
    plsc.subcore_barrier()

    pltpu.sync_copy(alpha_hbm, atbl)

    def issue_idx(c, p):
        b = base + c * K
        pltpu.async_copy(src_hbm.at[pl.ds(b, K)], srcbs[p], isems[p])
        pltpu.async_copy(dst_hbm.at[pl.ds(b, K)], dstbs[p], isems[p])

    def wait_idx(p):
        pltpu.make_async_copy(src_hbm.at[pl.ds(0, K)], srcbs[p], isems[p]).wait()
        pltpu.make_async_copy(dst_hbm.at[pl.ds(0, K)], dstbs[p], isems[p]).wait()

    def wait_scat(p):
        for h in range(NH):
            pltpu.make_async_copy(
                evbss[p][h], dsh.at[ixbss[p][h]], ssems[p]).wait()

    def step(c, p, first, last):
        wait_idx(p)
        if not first:
            wait_scat(p)

        def comp(g, _):
            sv6 = srcbs[p][pl.ds(g * 16, 16)] * 6
            dv = dstbs[p][pl.ds(g * 16, 16)]
            dv6 = dv * 6
            dv4 = dv * 4
            for h in range(NH):
                a_s = plsc.load_gather(atbl, [sv6 + h])
                a_d = plsc.load_gather(atbl, [dv6 + (3 + h)])
                z = a_s + a_d
                ev = jnp.exp(jnp.maximum(z, NEG * z))
                ebufs[h][pl.ds(c * K + g * 16, 16)] = ev
                evbss[p][h][pl.ds(g * 16, 16)] = ev
                ixbss[p][h][pl.ds(g * 16, 16)] = dv4 + h
            return 0
        lax.fori_loop(0, NG, comp, 0)
        for h in range(NH):
            pltpu.async_copy(evbss[p][h], dsh.at[ixbss[p][h]], ssems[p],
                             add=True)
        if not last:
            @pl.when(c + 2 < NCHUNK)
            def _():
                issue_idx(c + 2, p)

    issue_idx(0, 0)
    issue_idx(1, 1)
    step(0, 0, True, False)
    step(1, 1, True, False)

    def loop(i, _):
        c = 2 * i
        step(c, 0, False, False)
        step(c + 1, 1, False, False)
        return 0

    lax.fori_loop(1, (NCHUNK - 1) // 2, loop, 0)
    step(NCHUNK - 1, 0, False, True)
    wait_scat(1)
    wait_scat(0)

    for h in range(NH):
        pltpu.sync_copy(ebufs[h], ehbms[h].at[pl.ds(base, EPW)])

    plsc.subcore_barrier()
    pltpu.sync_copy(dsh.at[pl.ds(sid * DWPT, DWPT)],
                    denp_hbm.at[pl.ds(cid * DW + sid * DWPT, DWPT)])


def _pass1(src, dst, alpha_flat):
    f = functools.partial(
        pl.kernel,
        out_type=(
            jax.ShapeDtypeStruct((E,), jnp.float32),
            jax.ShapeDtypeStruct((E,), jnp.float32),
            jax.ShapeDtypeStruct((E,), jnp.float32),
            jax.ShapeDtypeStruct((NC * DW,), jnp.float32),
        ),
        mesh=_mesh(),
        scratch_types=[
            pltpu.VMEM((N * 6,), jnp.float32),
            [pltpu.VMEM((K,), jnp.int32) for _ in range(2)],
            [pltpu.VMEM((K,), jnp.int32) for _ in range(2)],
            pltpu.VMEM((EPW,), jnp.float32),
            pltpu.VMEM((EPW,), jnp.float32),
            pltpu.VMEM((EPW,), jnp.float32),
            [[pltpu.VMEM((PPS * 16,), jnp.float32) for _ in range(3)]
             for _ in range(2)],
            [[pltpu.VMEM((PPS * 16,), jnp.int32) for _ in range(3)]
             for _ in range(2)],
            pltpu.VMEM((DWPT // 2,), jnp.float32),
            pltpu.VMEM_SHARED((DW,), jnp.float32),
            [pltpu.SemaphoreType.DMA for _ in range(2)],
            [pltpu.SemaphoreType.DMA for _ in range(2)],
        ],
        compiler_params=pltpu.CompilerParams(needs_layout_passes=False),
    )(_pass1_body)
    return f(src, dst, alpha_flat)


def _rden_body(denp_ref, rden_ref):
    s = denp_ref[0:DW] + denp_ref[DW:2 * DW]
    rden_ref[...] = 1.0 / (s + 1e-16)


def _rden(denp):
    return pl.pallas_call(
        _rden_body,
        out_shape=jax.ShapeDtypeStruct((DW,), jnp.float32),
    )(denp)


def _pass2_body(rec_hbm, hcat_hbm, etcat_hbm, rden_hbm, outp_hbm,
                rtbl, recbs, sdst, w0b, w1b, w2b,
                hbufs, ewbufs, msgbuf, osh, isems, gsems, ssem):
    cid = lax.axis_index("c")
    sid = lax.axis_index("s")
    wid = sid * NC + cid
    rbase = wid * NCHUNK
    wbufs = (w0b, w1b, w2b)
    RW = 6 * K

    zero16 = jnp.zeros((16,), jnp.float32)

    def zmsg():
        def zrow(i, _):
            for q in range(HD // 16):
                msgbuf[i, pl.ds(q * 16, 16)] = zero16
            return 0
        lax.fori_loop(0, K, zrow, 0)

    def zosh():
        for j in range(RPT // K):
            pltpu.sync_copy(msgbuf, osh.at[pl.ds(sid * RPT + j * K, K)])

    zmsg()
    zosh()
    plsc.subcore_barrier()

    pltpu.sync_copy(rden_hbm, rtbl)

    def issue_idx(c, p):
        pltpu.async_copy(rec_hbm.at[pl.ds((rbase + c) * RW, RW)],
                         recbs[p], isems[p])

    def wait_idx(p, half):
        pltpu.make_async_copy(rec_hbm.at[pl.ds(0, RW)],
                              recbs[p], isems[p]).wait()
        srow = half * N
        erow = half * EV

        def fix(g, _):
            recbs[p][pl.ds(g * 16, 16)] = recbs[p][pl.ds(g * 16, 16)] + srow
            recbs[p][pl.ds(2 * K + g * 16, 16)] = \
                recbs[p][pl.ds(2 * K + g * 16, 16)] + erow
            return 0
        lax.fori_loop(0, NG, fix, 0)

    def issue_gath(p):
        pltpu.async_copy(hcat_hbm.at[recbs[p].at[pl.ds(0, K)]],
                         hbufs[p], gsems[p])
        pltpu.async_copy(etcat_hbm.at[recbs[p].at[pl.ds(2 * K, K)]],
                         ewbufs[p], gsems[p])

    def wait_gath(p):
        pltpu.make_async_copy(hcat_hbm.at[recbs[p].at[pl.ds(0, K)]],
                              hbufs[p], gsems[p]).wait()
        pltpu.make_async_copy(etcat_hbm.at[recbs[p].at[pl.ds(2 * K, K)]],
                              ewbufs[p], gsems[p]).wait()

    def wait_scat():
        pltpu.make_async_copy(msgbuf, osh.at[sdst], ssem).wait()

    def step(c, p, first, last, half):
        wait_gath(p)

        def wcomp(g, _):
            dv = recbs[p][pl.ds(K + g * 16, 16)]
            dv4 = dv * 4
            for h in range(NH):
                rd = plsc.load_gather(rtbl, [dv4 + h])
                eh = plsc.bitcast(
                    recbs[p][pl.ds((3 + h) * K + g * 16, 16)], jnp.float32)
                wbufs[h][pl.ds(g * 16, 16)] = eh * rd * jnp.float32(1.0 / NH)
            return 0
        lax.fori_loop(0, NG, wcomp, 0)
        if not first:
            wait_scat()

        def cpd(g, _):
            sdst[pl.ds(g * 16, 16)] = recbs[p][pl.ds(K + g * 16, 16)]
            return 0
        lax.fori_loop(0, NG, cpd, 0)
        if not last:
            @pl.when(c + 3 < NCHUNK)
            def _():
                issue_idx(c + 3, p)
            pn = (p + 2) % 3
            wait_idx(pn, half)
            issue_gath(pn)
        hbuf = hbufs[p]
        ewbuf = ewbufs[p]

        def mcomp(g, _):
            wv0 = w0b[pl.ds(g * 16, 16)]
            wv1 = w1b[pl.ds(g * 16, 16)]
            wv2 = w2b[pl.ds(g * 16, 16)]
            for kk in range(16):
                k = g * 16 + kk
                w0 = wv0[kk]
                w1 = wv1[kk]
                w2 = wv2[kk]
                hv = [hbuf[k, pl.ds(h * HW, HW)] for h in range(NH)]
                ev = ewbuf[k, pl.ds(0, HW)]
                me = (plsc.bitcast(hv[0] << 16, jnp.float32) * w0
                      + plsc.bitcast(hv[1] << 16, jnp.float32) * w1
                      + plsc.bitcast(hv[2] << 16, jnp.float32) * w2) \
                    * plsc.bitcast(ev << 16, jnp.float32)
                mo = (plsc.bitcast(hv[0] & BMSK, jnp.float32) * w0
                      + plsc.bitcast(hv[1] & BMSK, jnp.float32) * w1
                      + plsc.bitcast(hv[2] & BMSK, jnp.float32) * w2) \
                    * plsc.bitcast(ev & BMSK, jnp.float32)
                msgbuf[k, pl.ds(0, 16)] = me
                msgbuf[k, pl.ds(16, 16)] = mo
            return 0
        lax.fori_loop(0, NG, mcomp, 0)
        pltpu.async_copy(msgbuf, osh.at[sdst], ssem, add=True)

    def run_half(half, _):
        issue_idx(0, 0)
        issue_idx(1, 1)
        issue_idx(2, 2)
        wait_idx(0, half)
        issue_gath(0)
        wait_idx(1, half)
        issue_gath(1)
        step(0, 0, True, False, half)
        step(1, 1, False, False, half)
        step(2, 2, False, False, half)

        def loop(i, _):
            c = 3 * i
            step(c, 0, False, False, half)
            step(c + 1, 1, False, False, half)
            step(c + 2, 2, False, False, half)
            return 0

        lax.fori_loop(1, (NCHUNK - 2) // 3, loop, 0)
        step(NCHUNK - 2, 0, False, True, half)
        step(NCHUNK - 1, 1, False, True, half)
        wait_scat()

        plsc.subcore_barrier()
        pltpu.sync_copy(
            osh.at[pl.ds(sid * RPT, RPT)],
            outp_hbm.at[pl.ds(half * (NC * NP) + cid * NP + sid * RPT, RPT)])
        zmsg()
        zosh()
        plsc.subcore_barrier()
        return 0

    lax.fori_loop(0, NQ, run_half, 0)


def _pass2(rec, Hcat, etcat, rden):
    f = functools.partial(
        pl.kernel,
        out_type=jax.ShapeDtypeStruct((NQ * NC * NP, HD), jnp.float32),
        mesh=_mesh(),
        scratch_types=[
            pltpu.VMEM((DW,), jnp.float32),
            [pltpu.VMEM((6 * K,), jnp.int32) for _ in range(3)],
            pltpu.VMEM((K,), jnp.int32),
            pltpu.VMEM((K,), jnp.float32),
            pltpu.VMEM((K,), jnp.float32),
            pltpu.VMEM((K,), jnp.float32),
            [pltpu.VMEM((K, NH * HW), jnp.int32) for _ in range(3)],
            [pltpu.VMEM((K, HW), jnp.int32) for _ in range(3)],
            pltpu.VMEM((K, HD), jnp.float32),
            pltpu.VMEM_SHARED((NP, HD), jnp.float32),
            [pltpu.SemaphoreType.DMA for _ in range(3)],
            [pltpu.SemaphoreType.DMA for _ in range(3)],
            pltpu.SemaphoreType.DMA,
        ],
        compiler_params=pltpu.CompilerParams(
            needs_layout_passes=False, use_tc_tiling_on_sc=False),
    )(_pass2_body)
    return f(rec, Hcat, etcat, rden)


def _final_body(outp_ref, out_ref):
    parts = []
    for q in range(NQ):
        b = q * NC * NP
        parts.append(outp_ref[b:b + N, :] + outp_ref[b + NP:b + NP + N, :])
    out_ref[...] = jnp.concatenate(parts, axis=1)


def _final(outp):
    return pl.pallas_call(
        _final_body,
        out_shape=jax.ShapeDtypeStruct((N, D), jnp.float32),
    )(outp)


def kernel(x, edge_index, edge_weight, W_lin, edge_table, W_heads, a_src, a_dst):
    ei = edge_index.astype(jnp.int32)
    src = ei[0]
    dst = ei[1]
    ewi = edge_weight.astype(jnp.int32)
    Hcat, alpha = _dense(x, W_lin, W_heads, a_src, a_dst)
    etcat = _etsplit(edge_table)
    e0, e1, e2, denp = _pass1(src, dst, alpha.reshape(N * 6))
    rden = _rden(denp)
    hq = Hcat.reshape(NH, N, NQ, HD).transpose(2, 1, 0, 3)
    hi32 = lax.bitcast_convert_type(
        hq.reshape(NQ * N, NH * HW, 2), jnp.int32)
    eq = etcat.reshape(EV, NQ, HD).transpose(1, 0, 2)
    ei32 = lax.bitcast_convert_type(
        eq.reshape(NQ * EV, HW, 2), jnp.int32)
    bc = lax.bitcast_convert_type
    rec = jnp.stack(
        [a.reshape(NW * NCHUNK, K) for a in
         (src, dst, ewi, bc(e0, jnp.int32), bc(e1, jnp.int32),
          bc(e2, jnp.int32))],
        axis=1).reshape(NW * NCHUNK * 6 * K)
    outp = _pass2(rec, hi32, ei32, rden)
    out_perm = _final(outp)
    return out_perm.reshape(N, NQ, 2, 16).transpose(0, 1, 3, 2).reshape(N, D)
